# baseline (speedup 1.0000x reference)
import jax
import jax.numpy as jnp
from jax.experimental import pallas as pl
from jax.experimental.pallas import tpu as pltpu

B = 16
C = 256
BC = B * C
S_TOTAL = 5440


def _kern(f0, f1, f2, f3, emb, out_ref):
    i = pl.program_id(0)

    def level(x, h, w, row0, lvl):
        e = emb[lvl][None, :]
        for hi in range(h):
            out_ref[row0 + hi * w : row0 + (hi + 1) * w, :] = x[0, :, hi, :].T + e

    @pl.when(i == 0)
    def _():
        level(f0, 64, 64, 0, 0)

    @pl.when(i == 1)
    def _():
        level(f1, 32, 32, 0, 1)
        level(f2, 16, 16, 1024, 2)
        level(f3, 8, 8, 1280, 3)


def kernel(feats_0, feats_1, feats_2, feats_3, level_start_idx, spatial_shapes, embed_weight):
    in_specs = [
        pl.BlockSpec((1, C, 64, 64), lambda i, j: (jnp.where(i == 0, j, B - 1), 0, 0, 0)),
        pl.BlockSpec((1, C, 32, 32), lambda i, j: (jnp.where(i == 0, 0, j), 0, 0, 0)),
        pl.BlockSpec((1, C, 16, 16), lambda i, j: (jnp.where(i == 0, 0, j), 0, 0, 0)),
        pl.BlockSpec((1, C, 8, 8), lambda i, j: (jnp.where(i == 0, 0, j), 0, 0, 0)),
        pl.BlockSpec((4, C), lambda i, j: (0, 0)),
    ]
    out = pl.pallas_call(
        _kern,
        grid=(2, B),
        in_specs=in_specs,
        out_specs=pl.BlockSpec((4096, C), lambda i, j: (i, j)),
        out_shape=jax.ShapeDtypeStruct((S_TOTAL, BC), jnp.float32),
        compiler_params=pltpu.CompilerParams(
            dimension_semantics=("arbitrary", "arbitrary"),
        ),
    )(feats_0, feats_1, feats_2, feats_3, embed_weight)
    return out.reshape(S_TOTAL, B, C)


# speedup vs baseline: 1.2418x; 1.2418x over previous
import jax
import jax.numpy as jnp
from jax.experimental import pallas as pl
from jax.experimental.pallas import tpu as pltpu

B = 16
C = 256
BC = B * C
S_TOTAL = 5440


def _kern(f0, f1, f2, f3, emb, out_ref):
    i = pl.program_id(0)

    def level(x, h, w, row0, lvl):
        e = emb[lvl][None, :]
        flat = x[...].reshape(C, h * w)
        out_ref[row0 : row0 + h * w, :] = flat.T + e

    @pl.when(i == 0)
    def _():
        level(f0, 64, 64, 0, 0)

    @pl.when(i == 1)
    def _():
        level(f1, 32, 32, 0, 1)
        level(f2, 16, 16, 1024, 2)
        level(f3, 8, 8, 1280, 3)


def kernel(feats_0, feats_1, feats_2, feats_3, level_start_idx, spatial_shapes, embed_weight):
    in_specs = [
        pl.BlockSpec((1, C, 64, 64), lambda i, j: (jnp.where(i == 0, j, B - 1), 0, 0, 0)),
        pl.BlockSpec((1, C, 32, 32), lambda i, j: (jnp.where(i == 0, 0, j), 0, 0, 0)),
        pl.BlockSpec((1, C, 16, 16), lambda i, j: (jnp.where(i == 0, 0, j), 0, 0, 0)),
        pl.BlockSpec((1, C, 8, 8), lambda i, j: (jnp.where(i == 0, 0, j), 0, 0, 0)),
        pl.BlockSpec((4, C), lambda i, j: (0, 0)),
    ]
    out = pl.pallas_call(
        _kern,
        grid=(2, B),
        in_specs=in_specs,
        out_specs=pl.BlockSpec((4096, C), lambda i, j: (i, j)),
        out_shape=jax.ShapeDtypeStruct((S_TOTAL, BC), jnp.float32),
        compiler_params=pltpu.CompilerParams(
            dimension_semantics=("arbitrary", "arbitrary"),
        ),
    )(feats_0, feats_1, feats_2, feats_3, embed_weight)
    return out.reshape(S_TOTAL, B, C)
